# Initial kernel scaffold; baseline (speedup 1.0000x reference)
import jax
import jax.numpy as jnp
from jax.experimental import pallas as pl


def kernel(x, Wg, W1, W2, ln_w, ln_b):
    raise NotImplementedError("write your pallas kernel here")


# speedup vs baseline: 1.7490x; 1.7490x over previous
import functools
import jax
import jax.numpy as jnp
from jax.experimental import pallas as pl
from jax.experimental.pallas import tpu as pltpu

_N, _D, _H, _E = 2048, 1024, 512, 8
_BN = 256
_NI = _N // _BN


def _moe_kernel(x_ref, wg_ref, w1_ref, w2_ref, lnw_ref, lnb_ref, out_ref, acc_ref):
    e = pl.program_id(0)
    i = pl.program_id(1)
    x = x_ref[...]
    wg = wg_ref[...]

    logits = jax.lax.dot_general(x, wg, (((1,), (1,)), ((), ())),
                                 preferred_element_type=jnp.float32)
    m = jnp.max(logits, axis=-1, keepdims=True)
    p = jnp.exp(logits - m)
    lane = jax.lax.broadcasted_iota(jnp.int32, p.shape, 1)
    p0 = jnp.max(p, axis=-1, keepdims=True)
    e0 = jnp.min(jnp.where(p == p0, lane, _E), axis=-1, keepdims=True)
    p_m = jnp.where(lane == e0, -jnp.inf, p)
    p1 = jnp.max(p_m, axis=-1, keepdims=True)
    e1 = jnp.min(jnp.where(p_m == p1, lane, _E), axis=-1, keepdims=True)
    s = p0 + p1
    ce = jnp.where(e0 == e, p0 / s, 0.0) + jnp.where(e1 == e, p1 / s, 0.0)

    w1 = w1_ref[0]
    w2 = w2_ref[0]
    h = jax.lax.dot_general(x, w1, (((1,), (1,)), ((), ())),
                            preferred_element_type=jnp.float32)
    mu = jnp.mean(h, axis=-1, keepdims=True)
    var = jnp.mean((h - mu) ** 2, axis=-1, keepdims=True)
    hn = (h - mu) * jax.lax.rsqrt(var + 1e-5)
    hn = hn * lnw_ref[0] + lnb_ref[0]
    a = hn * 0.5 * (1.0 + jax.lax.erf(hn * 0.7071067811865476))
    y = jax.lax.dot_general(a, w2, (((1,), (1,)), ((), ())),
                            preferred_element_type=jnp.float32)
    val = ce * y

    sl = pl.ds(i * _BN, _BN)

    @pl.when(e == 0)
    def _():
        acc_ref[sl, :] = val

    @pl.when(e != 0)
    def _():
        acc_ref[sl, :] = acc_ref[sl, :] + val

    @pl.when(e == _E - 1)
    def _():
        out_ref[...] = acc_ref[sl, :]


def kernel(x, Wg, W1, W2, ln_w, ln_b):
    grid = (_E, _NI)
    return pl.pallas_call(
        _moe_kernel,
        grid=grid,
        in_specs=[
            pl.BlockSpec((_BN, _D), lambda e, i: (i, 0)),
            pl.BlockSpec((_E, _D), lambda e, i: (0, 0)),
            pl.BlockSpec((1, _H, _D), lambda e, i: (e, 0, 0)),
            pl.BlockSpec((1, _D, _H), lambda e, i: (e, 0, 0)),
            pl.BlockSpec((1, 1, _H), lambda e, i: (e, 0, 0)),
            pl.BlockSpec((1, 1, _H), lambda e, i: (e, 0, 0)),
        ],
        out_specs=pl.BlockSpec(
            (_BN, _D), lambda e, i: (jnp.where(e == _E - 1, i, 0), 0)),
        out_shape=jax.ShapeDtypeStruct((_N, _D), jnp.float32),
        scratch_shapes=[pltpu.VMEM((_N, _D), jnp.float32)],
    )(x, Wg, W1, W2, ln_w.reshape(_E, 1, _H), ln_b.reshape(_E, 1, _H))
